# baseline (speedup 1.0000x reference)
import jax
import jax.numpy as jnp
from jax.experimental import pallas as pl
from jax.experimental.pallas import tpu as pltpu

_GRID = 25
_E_ROWS = 40000 // _GRID


def _copy_body(e_ref, oe_ref):
    oe_ref[...] = e_ref[...]


def kernel(x, edge_index, edge_attr):
    del edge_index
    n_edges, d_edge = edge_attr.shape
    e2 = edge_attr.reshape(n_edges * d_edge // 128, 128)
    out_e = pl.pallas_call(
        _copy_body,
        grid=(_GRID,),
        in_specs=[pl.BlockSpec((_E_ROWS, 128), lambda i: (i, 0))],
        out_specs=pl.BlockSpec((_E_ROWS, 128), lambda i: (i, 0)),
        out_shape=jax.ShapeDtypeStruct(e2.shape, e2.dtype),
        compiler_params=pltpu.CompilerParams(
            dimension_semantics=("arbitrary",),
        ),
    )(e2)
    return (x, out_e.reshape(n_edges, d_edge))
